# baseline (speedup 1.0000x reference)
import functools

import jax
import jax.numpy as jnp
from jax import lax
from jax.experimental import pallas as pl
from jax.experimental.pallas import tpu as pltpu
from jax.experimental.pallas import tpu_sc as plsc

_INFO = plsc.get_sparse_core_info()
_NC, _NS = _INFO.num_cores, _INFO.num_subcores
_NW = _NC * _NS


def _make_gather(n: int, m: int, v_len: int):
    assert n % _NW == 0
    rows_per_w = n // _NW
    mesh = plsc.VectorSubcoreMesh(core_axis_name="c", subcore_axis_name="s")

    chunk = 128
    n_chunks = rows_per_w // chunk
    assert rows_per_w % chunk == 0

    @functools.partial(
        pl.kernel,
        mesh=mesh,
        out_type=jax.ShapeDtypeStruct((n, m), jnp.float32),
        scratch_types=[
            pltpu.VMEM((chunk, m), jnp.int32),
            pltpu.VMEM((chunk, m), jnp.float32),
            pltpu.VMEM_SHARED((v_len,), jnp.float32),
            pltpu.SemaphoreType.DMA,
        ],
        compiler_params=pltpu.CompilerParams(use_tc_tiling_on_sc=True),
    )
    def gather_kernel(idx_hbm, table_hbm, out_hbm, idx_v, rows_v, tbl_s, sem):
        sid = lax.axis_index("s")
        wid = sid * _NC + lax.axis_index("c")
        base = wid * rows_per_w

        @pl.when(sid == 0)
        def _():
            pltpu.sync_copy(table_hbm, tbl_s)

        plsc.subcore_barrier()

        def do_chunk(c, carry):
            cbase = base + c * chunk
            pltpu.sync_copy(idx_hbm.at[pl.ds(cbase, chunk), :], idx_v)

            def fire(r, cy):
                pltpu.make_async_copy(
                    tbl_s.at[idx_v.at[r]], rows_v.at[r], sem
                ).start()
                return cy

            lax.fori_loop(0, chunk, fire, 0)

            def drain(r, cy):
                pltpu.make_async_copy(
                    tbl_s.at[idx_v.at[r]], rows_v.at[r], sem
                ).wait()
                return cy

            lax.fori_loop(0, chunk, drain, 0)
            pltpu.sync_copy(rows_v, out_hbm.at[pl.ds(cbase, chunk), :])
            return carry

        lax.fori_loop(0, n_chunks, do_chunk, 0)

    return gather_kernel


@jax.jit
def kernel(idx, v):
    n, m = idx.shape
    return _make_gather(n, m, v.shape[0])(idx.astype(jnp.int32), v)


# speedup vs baseline: 1.3390x; 1.3390x over previous
import functools

import jax
import jax.numpy as jnp
from jax import lax
from jax.experimental import pallas as pl
from jax.experimental.pallas import tpu as pltpu
from jax.experimental.pallas import tpu_sc as plsc

_INFO = plsc.get_sparse_core_info()
_NC, _NS = _INFO.num_cores, _INFO.num_subcores
_NW = _NC * _NS


def _make_gather(n_rows: int, n_cols: int, v_len: int):
    col_chunk = 128
    assert n_cols % (_NW * col_chunk) == 0
    cols_per_w = n_cols // _NW
    n_passes = cols_per_w // col_chunk
    mesh = plsc.VectorSubcoreMesh(core_axis_name="c", subcore_axis_name="s")

    @functools.partial(
        pl.kernel,
        mesh=mesh,
        out_type=jax.ShapeDtypeStruct((n_rows, n_cols), jnp.float32),
        scratch_types=[
            pltpu.VMEM((n_rows, col_chunk), jnp.int32),
            pltpu.VMEM((n_rows, col_chunk), jnp.float32),
            pltpu.VMEM_SHARED((v_len,), jnp.float32),
            pltpu.SemaphoreType.DMA,
        ],
    )
    def gather_kernel(idx_hbm, table_hbm, out_hbm, idx_v, rows_v, tbl_s, sem):
        sid = lax.axis_index("s")
        wid = sid * _NC + lax.axis_index("c")

        @pl.when(sid == 0)
        def _():
            pltpu.sync_copy(table_hbm, tbl_s)

        plsc.subcore_barrier()

        def do_pass(p, carry):
            cbase = wid * cols_per_w + p * col_chunk
            pltpu.sync_copy(idx_hbm.at[:, pl.ds(cbase, col_chunk)], idx_v)

            def fire(r, cy):
                pltpu.make_async_copy(
                    tbl_s.at[idx_v.at[r]], rows_v.at[r], sem
                ).start()
                return cy

            lax.fori_loop(0, n_rows, fire, 0)

            def drain(r, cy):
                pltpu.make_async_copy(
                    tbl_s.at[idx_v.at[r]], rows_v.at[r], sem
                ).wait()
                return cy

            lax.fori_loop(0, n_rows, drain, 0)
            pltpu.sync_copy(rows_v, out_hbm.at[:, pl.ds(cbase, col_chunk)])
            return carry

        lax.fori_loop(0, n_passes, do_pass, 0)

    return gather_kernel


@jax.jit
def kernel(idx, v):
    n, m = idx.shape
    idx_t = idx.astype(jnp.int32).T
    out_t = _make_gather(m, n, v.shape[0])(idx_t, v)
    return out_t.T
